# Initial kernel scaffold; baseline (speedup 1.0000x reference)
import jax
import jax.numpy as jnp
from jax.experimental import pallas as pl


def kernel(x_question_id, x_part, x_tag, x_interaction, W_question_id, W_part, W_tag, W_interaction):
    raise NotImplementedError("write your pallas kernel here")


# speedup vs baseline: 1.0021x; 1.0021x over previous
import jax
import jax.numpy as jnp
from jax import lax
from jax.experimental import pallas as pl
from jax.experimental.pallas import tpu as pltpu, tpu_sc as plsc

B, L, D = 4096, 200, 16
N = B * L
NC, NS = 2, 16
NW = NC * NS
NPW = N // NW
CK = 128
NCH = NPW // CK


def _embed_body(xq, xp, xt, xi, wq, wp, wt, wi, out, idx_v, rows_v, gsem):
    wid = lax.axis_index("s") * NC + lax.axis_index("c")
    for f, (x, w) in enumerate(((xq, wq), (xp, wp), (xt, wt), (xi, wi))):
        pltpu.sync_copy(x.at[wid], idx_v)

        @pl.loop(0, NCH)
        def chunk_loop(j):
            pltpu.async_copy(w.at[idx_v.at[j]], rows_v, gsem).wait()
            base = wid * NPW + j * CK
            pltpu.sync_copy(rows_v, out.at[pl.ds(base, CK), pl.ds(f * D, D)])


@jax.jit
def kernel(x_question_id, x_part, x_tag, x_interaction,
           W_question_id, W_part, W_tag, W_interaction):
    xs = [x.reshape(NW, NCH, CK)
          for x in (x_question_id, x_part, x_tag, x_interaction)]
    mesh = plsc.VectorSubcoreMesh(core_axis_name="c", subcore_axis_name="s",
                                  num_cores=NC, num_subcores=NS)
    out = pl.kernel(
        _embed_body,
        out_type=jax.ShapeDtypeStruct((N, 4 * D), jnp.float32),
        mesh=mesh,
        scratch_types=[
            pltpu.VMEM((NCH, CK), jnp.int32),
            pltpu.VMEM((CK, D), jnp.float32),
            pltpu.SemaphoreType.DMA,
        ],
        compiler_params=pltpu.CompilerParams(use_tc_tiling_on_sc=False),
    )(xs[0], xs[1], xs[2], xs[3], W_question_id, W_part, W_tag, W_interaction)
    return out.reshape(B, L, 4 * D)


# speedup vs baseline: 1.0262x; 1.0241x over previous
import jax
import jax.numpy as jnp
from jax import lax
from jax.experimental import pallas as pl
from jax.experimental.pallas import tpu as pltpu, tpu_sc as plsc

B, L, D = 4096, 200, 16
N = B * L
NC, NS = 2, 16
NW = NC * NS
NPW = N // NW
CK = 1280
NCH = NPW // CK


def _embed_body(xq, xp, xt, xi, wq, wp, wt, wi, out,
                idx_v, rows0, rows1, gsem0, gsem1):
    wid = lax.axis_index("s") * NC + lax.axis_index("c")
    bufs = (rows0, rows1)
    sems = (gsem0, gsem1)
    for f, (x, w) in enumerate(((xq, wq), (xp, wp), (xt, wt), (xi, wi))):
        pltpu.sync_copy(x.at[wid], idx_v)
        pltpu.async_copy(w.at[idx_v.at[0]], rows0, gsem0)

        @pl.loop(0, NCH, step=2)
        def chunk_loop(j0):
            for u in range(2):
                j = j0 + u
                buf, sem = bufs[u], sems[u]
                pltpu.make_async_copy(w.at[idx_v.at[j]], buf, sem).wait()

                @pl.when(j + 1 < NCH)
                def _fire_next():
                    pltpu.async_copy(w.at[idx_v.at[j + 1]],
                                     bufs[1 - u], sems[1 - u])

                base = wid * NPW + j * CK
                pltpu.sync_copy(buf, out.at[pl.ds(base, CK), pl.ds(f * D, D)])


@jax.jit
def kernel(x_question_id, x_part, x_tag, x_interaction,
           W_question_id, W_part, W_tag, W_interaction):
    xs = [x.reshape(NW, NCH, CK)
          for x in (x_question_id, x_part, x_tag, x_interaction)]
    mesh = plsc.VectorSubcoreMesh(core_axis_name="c", subcore_axis_name="s",
                                  num_cores=NC, num_subcores=NS)
    out = pl.kernel(
        _embed_body,
        out_type=jax.ShapeDtypeStruct((N, 4 * D), jnp.float32),
        mesh=mesh,
        scratch_types=[
            pltpu.VMEM((NCH, CK), jnp.int32),
            pltpu.VMEM((CK, D), jnp.float32),
            pltpu.VMEM((CK, D), jnp.float32),
            pltpu.SemaphoreType.DMA,
            pltpu.SemaphoreType.DMA,
        ],
        compiler_params=pltpu.CompilerParams(use_tc_tiling_on_sc=False),
    )(xs[0], xs[1], xs[2], xs[3], W_question_id, W_part, W_tag, W_interaction)
    return out.reshape(B, L, 4 * D)


# speedup vs baseline: 1.0267x; 1.0005x over previous
import jax
import jax.numpy as jnp
from jax import lax
from jax.experimental import pallas as pl
from jax.experimental.pallas import tpu as pltpu, tpu_sc as plsc

B, L, D = 4096, 200, 16
N = B * L
NC, NS = 2, 16
NW = NC * NS
NPW = N // NW
CK = 320
NCH = NPW // CK
NBUF = 8


def _embed_body(xq, xp, xt, xi, wq, wp, wt, wi, out, idx_v, *scr):
    rowsb = scr[:NBUF]
    gsem = scr[NBUF:]
    wid = lax.axis_index("s") * NC + lax.axis_index("c")

    for f, (x, w) in enumerate(((xq, wq), (xp, wp), (xt, wt), (xi, wi))):
        pltpu.sync_copy(x.at[wid], idx_v)
        for u in range(NBUF):
            pltpu.async_copy(w.at[idx_v.at[u]], rowsb[u], gsem[u])

        @pl.loop(0, NCH, step=NBUF)
        def chunk_loop(j0):
            for u in range(NBUF):
                j = j0 + u
                pltpu.make_async_copy(w.at[idx_v.at[j]], rowsb[u],
                                      gsem[u]).wait()
                base = wid * NPW + j * CK
                pltpu.sync_copy(rowsb[u],
                                out.at[pl.ds(base, CK), pl.ds(f * D, D)])

                @pl.when(j + NBUF < NCH)
                def _refire():
                    pltpu.async_copy(w.at[idx_v.at[j + NBUF]],
                                     rowsb[u], gsem[u])


@jax.jit
def kernel(x_question_id, x_part, x_tag, x_interaction,
           W_question_id, W_part, W_tag, W_interaction):
    xs = [x.reshape(NW, NCH, CK)
          for x in (x_question_id, x_part, x_tag, x_interaction)]
    mesh = plsc.VectorSubcoreMesh(core_axis_name="c", subcore_axis_name="s",
                                  num_cores=NC, num_subcores=NS)
    out = pl.kernel(
        _embed_body,
        out_type=jax.ShapeDtypeStruct((N, 4 * D), jnp.float32),
        mesh=mesh,
        scratch_types=(
            [pltpu.VMEM((NCH, CK), jnp.int32)]
            + [pltpu.VMEM((CK, D), jnp.float32) for _ in range(NBUF)]
            + [pltpu.SemaphoreType.DMA for _ in range(NBUF)]
        ),
        compiler_params=pltpu.CompilerParams(use_tc_tiling_on_sc=False),
    )(xs[0], xs[1], xs[2], xs[3], W_question_id, W_part, W_tag, W_interaction)
    return out.reshape(B, L, 4 * D)


# speedup vs baseline: 6.4387x; 6.2713x over previous
import jax
import jax.numpy as jnp
from jax import lax
from jax.experimental import pallas as pl
from jax.experimental.pallas import tpu as pltpu, tpu_sc as plsc

B, L, D = 4096, 200, 16
NC, NS = 2, 16
NW = NC * NS
NB = L // 8
SLOTS = 4


def _embed_body(xq, xp, xt, xi, wq, wp, wt, wi, out,
                idxb, rowsq, outb, tp, tt, ti,
                isem, qsem, wsem):
    wid = lax.axis_index("s") * NC + lax.axis_index("c")
    xs = (xq, xp, xt, xi)
    smalls = ((1, tp, wp), (2, tt, wt), (3, ti, wi))

    for _, tbl, w in smalls:
        pltpu.sync_copy(w, tbl)

    def fire_idx(tr, pp):
        for f in range(4):
            pltpu.async_copy(xs[f].at[tr, wid], idxb.at[pp, f], isem.at[pp])

    def wait_idx(tr, pp):
        for f in range(4):
            pltpu.make_async_copy(xs[f].at[tr, wid], idxb.at[pp, f],
                                  isem.at[pp]).wait()

    def qgather(tr, pp, sub):
        return pltpu.make_async_copy(wq.at[idxb.at[pp, 0, sub]],
                                     rowsq.at[pp, sub], qsem.at[pp, sub])

    def fire_qgathers(tr, pp):
        for sub in range(8):
            pltpu.async_copy(wq.at[idxb.at[pp, 0, sub]],
                             rowsq.at[pp, sub], qsem.at[pp, sub])

    def wblock(l, s):
        return pltpu.make_async_copy(outb.at[s], out.at[l, :, wid],
                                     wsem.at[s])

    fire_idx(0, 0)
    wait_idx(0, 0)
    fire_qgathers(0, 0)
    fire_idx(1, 1)

    @pl.loop(0, NB)
    def band(tr):
        p = tr % 2
        np_ = 1 - p

        @pl.when(tr + 1 < NB)
        def _fire_next_band():
            wait_idx(tr + 1, np_)
            fire_qgathers(tr + 1, np_)

        @pl.loop(0, 8)
        def subl(sub):
            l = tr * 8 + sub
            s = l % SLOTS

            @pl.when(l >= SLOTS)
            def _recycle_slot():
                wblock(l, s).wait()

            qgather(tr, p, sub).wait()
            rq = rowsq.at[p, sub]

            @pl.loop(0, 8)
            def bgrp(g):
                bvec = lax.iota(jnp.int32, 16) + g * 16
                for c in range(16):
                    cvec = jnp.full((16,), c, jnp.int32)
                    vals = plsc.load_gather(rq, [bvec, cvec])
                    outb[s, c // 8, c % 8, pl.ds(g * 16, 16)] = vals
                for f, tbl, _ in smalls:
                    idx16 = idxb[p, f, sub, pl.ds(g * 16, 16)]
                    for c in range(16):
                        cvec = jnp.full((16,), c, jnp.int32)
                        vals = plsc.load_gather(tbl, [idx16, cvec])
                        cc = f * 16 + c
                        outb[s, cc // 8, cc % 8, pl.ds(g * 16, 16)] = vals

            pltpu.async_copy(outb.at[s], out.at[l, :, wid], wsem.at[s])

        @pl.when(tr + 2 < NB)
        def _prefetch_idx():
            fire_idx(tr + 2, p)

    for s in range(SLOTS):
        wblock(L - SLOTS + s, (L - SLOTS + s) % SLOTS).wait()


@jax.jit
def kernel(x_question_id, x_part, x_tag, x_interaction,
           W_question_id, W_part, W_tag, W_interaction):
    xs = [x.T.reshape(NB, 8, NW, 128).transpose(0, 2, 1, 3)
          for x in (x_question_id, x_part, x_tag, x_interaction)]
    mesh = plsc.VectorSubcoreMesh(core_axis_name="c", subcore_axis_name="s",
                                  num_cores=NC, num_subcores=NS)
    out5 = pl.kernel(
        _embed_body,
        out_type=jax.ShapeDtypeStruct((L, 8, NW, 8, 128), jnp.float32),
        mesh=mesh,
        scratch_types=[
            pltpu.VMEM((2, 4, 8, 128), jnp.int32),
            pltpu.VMEM((2, 8, 128, D), jnp.float32),
            pltpu.VMEM((SLOTS, 8, 8, 128), jnp.float32),
            pltpu.VMEM((9, D), jnp.float32),
            pltpu.VMEM((189, D), jnp.float32),
            pltpu.VMEM((3, D), jnp.float32),
            pltpu.SemaphoreType.DMA((2,)),
            pltpu.SemaphoreType.DMA((2, 8)),
            pltpu.SemaphoreType.DMA((SLOTS,)),
        ],
        compiler_params=pltpu.CompilerParams(use_tc_tiling_on_sc=False,
                                             needs_layout_passes=False),
    )(xs[0], xs[1], xs[2], xs[3], W_question_id, W_part, W_tag, W_interaction)
    return out5.transpose(2, 4, 0, 1, 3).reshape(B, L, 4 * D)
